# Initial kernel scaffold; baseline (speedup 1.0000x reference)
import jax
import jax.numpy as jnp
from jax.experimental import pallas as pl


def kernel(x, node_type, W, b):
    raise NotImplementedError("write your pallas kernel here")


# speedup vs baseline: 2.9818x; 2.9818x over previous
import functools

import jax
import jax.numpy as jnp
from jax import lax
from jax.experimental import pallas as pl
from jax.experimental.pallas import tpu as pltpu


def _proj_block(nt_ref, x_ref, w_ref, b_ref, o_ref, *, T):
    xb = x_ref[...]
    ntb = nt_ref[...]
    R = xb.shape[0]
    onehot = (ntb == lax.broadcasted_iota(jnp.int32, (R, T), 1)).astype(jnp.float32)
    acc = jnp.zeros((R, w_ref.shape[1]), dtype=jnp.float32)
    for t in range(T):
        yt = lax.dot_general(
            xb, w_ref[t],
            dimension_numbers=(((1,), (1,)), ((), ())),
            preferred_element_type=jnp.float32,
        )
        bb = b_ref[t:t + 1, :]
        acc = acc + (yt + bb) * onehot[:, t:t + 1]
    o_ref[...] = acc


def kernel(x, node_type, W, b):
    N, D = x.shape
    T, H, _ = W.shape
    R = 1000
    assert N % R == 0
    NB = N // R
    nt2 = node_type.reshape(N, 1)

    grid_spec = pl.GridSpec(
        grid=(NB,),
        in_specs=[
            pl.BlockSpec((R, 1), lambda i: (i, 0)),
            pl.BlockSpec((R, D), lambda i: (i, 0)),
            pl.BlockSpec((T, H, D), lambda i: (0, 0, 0)),
            pl.BlockSpec((T, H), lambda i: (0, 0)),
        ],
        out_specs=pl.BlockSpec((R, H), lambda i: (i, 0)),
    )
    return pl.pallas_call(
        functools.partial(_proj_block, T=T),
        grid_spec=grid_spec,
        out_shape=jax.ShapeDtypeStruct((N, H), jnp.float32),
        compiler_params=pltpu.CompilerParams(
            dimension_semantics=("arbitrary",),
        ),
    )(nt2, x, W, b)
